# baseline (speedup 1.0000x reference)
import functools

import jax
import jax.numpy as jnp
from jax import lax
from jax.experimental import pallas as pl
from jax.experimental.pallas import tpu as pltpu
from jax.experimental.pallas import tpu_sc as plsc

NP = 102400
ROWS_PER_TILE = NP // 16
NW = 32
BIG = 320
BT = 6400


def _sc_mesh():
  return plsc.VectorSubcoreMesh(core_axis_name="c", subcore_axis_name="s")


@functools.lru_cache(maxsize=None)
def _make_deg_kernel(groups):
  @functools.partial(
      pl.kernel,
      mesh=_sc_mesh(),
      out_type=jax.ShapeDtypeStruct((2, NP), jnp.float32),
      compiler_params=pltpu.CompilerParams(use_tc_tiling_on_sc=False),
      scratch_types=[
          pltpu.VMEM_SHARED((NP,), jnp.float32),
          pltpu.VMEM((2, 2, BIG), jnp.int32),
          pltpu.VMEM((BIG,), jnp.float32),
          pltpu.VMEM((640,), jnp.float32),
          pltpu.SemaphoreType.DMA,
          pltpu.SemaphoreType.DMA,
          pltpu.SemaphoreType.DMA,
      ],
  )
  def deg_kernel(edges_hbm, out_hbm, acc_sh, idx_v, ones_v, zbuf_v,
                 isem, ssem, zsem):
    cid = lax.axis_index("c")
    sid = lax.axis_index("s")
    wid = cid * 16 + sid
    @pl.loop(0, BIG // 16)
    def _ofill(i):
      ones_v[pl.ds(i * 16, 16)] = jnp.ones((16,), jnp.float32)

    @pl.loop(0, 40)
    def _zfill(i):
      zbuf_v[pl.ds(i * 16, 16)] = jnp.zeros((16,), jnp.float32)

    base = sid * ROWS_PER_TILE
    zcps = [pltpu.async_copy(zbuf_v, acc_sh.at[pl.ds(base + k * 640, 640)],
                             zsem) for k in range(ROWS_PER_TILE // 640)]
    for zcp in zcps:
      zcp.wait()
    plsc.subcore_barrier()
    pltpu.async_copy(edges_hbm.at[1, wid, 0], idx_v.at[0], isem)

    @pl.loop(0, groups, step=2)
    def _group(g):
      for b in range(2):
        cur = g + b
        nb = 1 - b
        pltpu.make_async_copy(
            edges_hbm.at[1, wid, cur], idx_v.at[b], isem).wait()
        nxt = jnp.minimum(cur + 1, groups - 1)
        pltpu.async_copy(edges_hbm.at[1, wid, nxt], idx_v.at[nb], isem)
        scps = []
        for j in range(2):
          scps.append(pltpu.async_copy(
              ones_v, acc_sh.at[idx_v.at[b, j]], ssem, add=True))
        for scp in scps:
          scp.wait()

    pltpu.make_async_copy(edges_hbm.at[1, wid, 0], idx_v.at[0], isem).wait()
    plsc.subcore_barrier()
    pltpu.sync_copy(acc_sh.at[pl.ds(base, ROWS_PER_TILE)],
                    out_hbm.at[cid, pl.ds(base, ROWS_PER_TILE)])

  return deg_kernel


@functools.lru_cache(maxsize=None)
def _make_prop_kernel(groups, feat):
  @functools.partial(
      pl.kernel,
      mesh=_sc_mesh(),
      out_type=jax.ShapeDtypeStruct((2, NP, feat), jnp.float32),
      compiler_params=pltpu.CompilerParams(use_tc_tiling_on_sc=False),
      scratch_types=[
          pltpu.VMEM_SHARED((NP, feat), jnp.float32),
          pltpu.VMEM((2, 2, BIG), jnp.int32),
          pltpu.VMEM((2, 2, BIG), jnp.int32),
          pltpu.VMEM((4, BIG, feat), jnp.float32),
          pltpu.SemaphoreType.DMA,
          pltpu.SemaphoreType.DMA,
          pltpu.SemaphoreType.DMA,
          pltpu.SemaphoreType.DMA,
          pltpu.SemaphoreType.DMA,
          pltpu.SemaphoreType.DMA,
          pltpu.SemaphoreType.DMA,
          pltpu.SemaphoreType.DMA,
          pltpu.SemaphoreType.DMA,
          pltpu.SemaphoreType.DMA,
      ],
  )
  def prop_kernel(g_hbm, edges_hbm, out_hbm,
                  acc_sh, sidx_v, didx_v, rows_v,
                  isem, gsem0, gsem1, gsem2, gsem3,
                  ssem0, ssem1, ssem2, ssem3, zsem):
    cid = lax.axis_index("c")
    sid = lax.axis_index("s")
    wid = cid * 16 + sid
    gsem = (gsem0, gsem1, gsem2, gsem3)
    ssem = (ssem0, ssem1, ssem2, ssem3)
    base = sid * ROWS_PER_TILE

    for p in range(4):
      @pl.loop(0, BIG)
      def _zfill(r, _p=p):
        rows_v[_p, r, :] = jnp.zeros((feat,), jnp.float32)

    zcps = []
    for k in range(ROWS_PER_TILE // BIG):
      zcps.append(pltpu.async_copy(
          rows_v.at[k % 4],
          acc_sh.at[pl.ds(base + k * BIG, BIG)], zsem))
    for zcp in zcps:
      zcp.wait()
    plsc.subcore_barrier()
    pltpu.async_copy(edges_hbm.at[0, wid, 0], sidx_v.at[0], isem)
    pltpu.async_copy(edges_hbm.at[1, wid, 0], didx_v.at[0], isem)

    @pl.loop(0, groups, step=2)
    def _group(g):
      def idx(v, k):
        return v.at[k // 2, k % 2]

      pltpu.make_async_copy(edges_hbm.at[0, wid, g], sidx_v.at[0], isem).wait()
      pltpu.make_async_copy(edges_hbm.at[1, wid, g], didx_v.at[0], isem).wait()
      pltpu.async_copy(edges_hbm.at[0, wid, g + 1], sidx_v.at[1], isem)
      pltpu.async_copy(edges_hbm.at[1, wid, g + 1], didx_v.at[1], isem)
      gcp = [None] * 4
      scp = [None] * 4
      for k in range(4):
        if k == 2:
          pltpu.make_async_copy(
              edges_hbm.at[0, wid, g + 1], sidx_v.at[1], isem).wait()
          pltpu.make_async_copy(
              edges_hbm.at[1, wid, g + 1], didx_v.at[1], isem).wait()
        gcp[k] = pltpu.async_copy(
            g_hbm.at[idx(sidx_v, k)], rows_v.at[k], gsem[k])
        if k >= 1:
          gcp[k - 1].wait()
          scp[k - 1] = pltpu.async_copy(
              rows_v.at[k - 1], acc_sh.at[idx(didx_v, k - 1)], ssem[k - 1],
              add=True)
      gcp[3].wait()
      scp[3] = pltpu.async_copy(
          rows_v.at[3], acc_sh.at[idx(didx_v, 3)], ssem[3], add=True)
      scp[0].wait()
      scp[1].wait()
      nxt = jnp.minimum(g + 2, groups - 2)
      pltpu.async_copy(edges_hbm.at[0, wid, nxt], sidx_v.at[0], isem)
      pltpu.async_copy(edges_hbm.at[1, wid, nxt], didx_v.at[0], isem)
      scp[2].wait()
      scp[3].wait()

    pltpu.make_async_copy(edges_hbm.at[0, wid, 0], sidx_v.at[0], isem).wait()
    pltpu.make_async_copy(edges_hbm.at[1, wid, 0], didx_v.at[0], isem).wait()
    plsc.subcore_barrier()
    pltpu.sync_copy(acc_sh.at[pl.ds(base, ROWS_PER_TILE)],
                    out_hbm.at[cid, pl.ds(base, ROWS_PER_TILE)])

  return prop_kernel


def _row_spec(feat):
  return pl.BlockSpec((BT, feat), lambda i: (i, 0))


def _pair_spec(*feat):
  if feat:
    return pl.BlockSpec((2, BT, feat[0]), lambda i: (0, i, 0))
  return pl.BlockSpec((2, BT), lambda i: (0, i))


def _full_spec(shape):
  return pl.BlockSpec(shape, lambda i: (0,) * len(shape))


def _tc1_body(d, x, w1, dinv_o, t1_o, g1_o):
  deg = (d[0, :] + d[1, :] + 1.0).reshape(BT, 1)
  dinv = lax.rsqrt(deg)
  dinv_o[...] = dinv
  t1 = jnp.dot(x[...], w1[...], preferred_element_type=jnp.float32)
  t1_o[...] = t1
  g1_o[...] = t1 * dinv


def _tc2_body(s, t1, dinv, b1, h1_o, g2_o):
  dv = dinv[...]
  agg = dv * (s[0] + s[1]) + dv * dv * t1[...] + b1[...]
  h1 = jnp.maximum(agg, 0.0)
  h1_o[...] = h1
  g2_o[...] = h1 * dv


def _tc3_body(s, h1, dinv, w2, b2, wl, bl, out_o):
  dv = dinv[...]
  agg = dv * (s[0] + s[1]) + dv * dv * h1[...]
  h2 = jnp.dot(agg, w2[...], preferred_element_type=jnp.float32) + b2[...]
  h2 = jnp.maximum(h2, 0.0)
  z = jnp.dot(h2, wl[...], preferred_element_type=jnp.float32) + bl[...]
  out_o[...] = jax.nn.sigmoid(z)


def kernel(x, edge_index, W1, b1, W2, b2, Wl, bl):
  n = x.shape[0]
  e = edge_index.shape[1]
  groups = -(-e // (NW * 2 * BIG))
  groups += groups % 2
  ep = NW * groups * 2 * BIG
  edges = jnp.pad(edge_index, ((0, 0), (0, ep - e)), constant_values=n)
  edges = edges.reshape(2, NW, groups, 2, BIG)
  xp = jnp.zeros((NP, 4), jnp.float32).at[:n].set(x)

  grid = (NP // BT,)

  degs = _make_deg_kernel(groups)(edges)

  dinv, t1, g1 = pl.pallas_call(
      _tc1_body,
      grid=grid,
      in_specs=[_pair_spec(), _row_spec(4), _full_spec((4, 16))],
      out_specs=[_row_spec(1), _row_spec(16), _row_spec(16)],
      out_shape=[jax.ShapeDtypeStruct((NP, 1), jnp.float32),
                 jax.ShapeDtypeStruct((NP, 16), jnp.float32),
                 jax.ShapeDtypeStruct((NP, 16), jnp.float32)],
  )(degs, xp, W1)

  s1 = _make_prop_kernel(groups, 16)(g1, edges)

  h1, g2 = pl.pallas_call(
      _tc2_body,
      grid=grid,
      in_specs=[_pair_spec(16), _row_spec(16), _row_spec(1),
                _full_spec((1, 16))],
      out_specs=[_row_spec(16), _row_spec(16)],
      out_shape=[jax.ShapeDtypeStruct((NP, 16), jnp.float32),
                 jax.ShapeDtypeStruct((NP, 16), jnp.float32)],
  )(s1, t1, dinv, b1.reshape(1, 16))

  s2 = _make_prop_kernel(groups, 16)(g2, edges)

  out = pl.pallas_call(
      _tc3_body,
      grid=grid,
      in_specs=[_pair_spec(16), _row_spec(16), _row_spec(1),
                _full_spec((16, 32)), _full_spec((1, 32)),
                _full_spec((32, 1)), _full_spec((1, 1))],
      out_specs=_row_spec(1),
      out_shape=jax.ShapeDtypeStruct((n, 1), jnp.float32),
  )(s2, h1, dinv, W2, b2.reshape(1, 32), Wl, bl.reshape(1, 1))

  return out


# speedup vs baseline: 1.3810x; 1.3810x over previous
import functools

import jax
import jax.numpy as jnp
from jax import lax
from jax.experimental import pallas as pl
from jax.experimental.pallas import tpu as pltpu
from jax.experimental.pallas import tpu_sc as plsc

NP = 102400
ROWS_PER_TILE = NP // 16
NW = 32
BIG = 256
BT = 6400


def _sc_mesh():
  return plsc.VectorSubcoreMesh(core_axis_name="c", subcore_axis_name="s")


@functools.lru_cache(maxsize=None)
def _make_deg_kernel(groups):
  @functools.partial(
      pl.kernel,
      mesh=_sc_mesh(),
      out_type=jax.ShapeDtypeStruct((2, NP), jnp.float32),
      compiler_params=pltpu.CompilerParams(use_tc_tiling_on_sc=False),
      scratch_types=[
          pltpu.VMEM_SHARED((NP,), jnp.float32),
          pltpu.VMEM((2, 2, BIG), jnp.int32),
          pltpu.VMEM((BIG,), jnp.float32),
          pltpu.VMEM((640,), jnp.float32),
          pltpu.SemaphoreType.DMA,
          pltpu.SemaphoreType.DMA,
          pltpu.SemaphoreType.DMA,
      ],
  )
  def deg_kernel(edges_hbm, out_hbm, acc_sh, idx_v, ones_v, zbuf_v,
                 isem, ssem, zsem):
    cid = lax.axis_index("c")
    sid = lax.axis_index("s")
    wid = cid * 16 + sid
    @pl.loop(0, BIG // 16)
    def _ofill(i):
      ones_v[pl.ds(i * 16, 16)] = jnp.ones((16,), jnp.float32)

    @pl.loop(0, 40)
    def _zfill(i):
      zbuf_v[pl.ds(i * 16, 16)] = jnp.zeros((16,), jnp.float32)

    base = sid * ROWS_PER_TILE
    zcps = [pltpu.async_copy(zbuf_v, acc_sh.at[pl.ds(base + k * 640, 640)],
                             zsem) for k in range(ROWS_PER_TILE // 640)]
    for zcp in zcps:
      zcp.wait()
    plsc.subcore_barrier()
    pltpu.async_copy(edges_hbm.at[1, wid, 0], idx_v.at[0], isem)

    @pl.loop(0, groups, step=2)
    def _group(g):
      for b in range(2):
        cur = g + b
        nb = 1 - b
        pltpu.make_async_copy(
            edges_hbm.at[1, wid, cur], idx_v.at[b], isem).wait()
        nxt = jnp.minimum(cur + 1, groups - 1)
        pltpu.async_copy(edges_hbm.at[1, wid, nxt], idx_v.at[nb], isem)
        scps = []
        for j in range(2):
          scps.append(pltpu.async_copy(
              ones_v, acc_sh.at[idx_v.at[b, j]], ssem, add=True))
        for scp in scps:
          scp.wait()

    pltpu.make_async_copy(edges_hbm.at[1, wid, 0], idx_v.at[0], isem).wait()
    plsc.subcore_barrier()
    pltpu.sync_copy(acc_sh.at[pl.ds(base, ROWS_PER_TILE)],
                    out_hbm.at[cid, pl.ds(base, ROWS_PER_TILE)])

  return deg_kernel


@functools.lru_cache(maxsize=None)
def _make_prop_kernel(groups, feat):
  @functools.partial(
      pl.kernel,
      mesh=_sc_mesh(),
      out_type=jax.ShapeDtypeStruct((2, NP, feat), jnp.float32),
      compiler_params=pltpu.CompilerParams(use_tc_tiling_on_sc=False),
      scratch_types=[
          pltpu.VMEM_SHARED((NP, feat), jnp.float32),
          pltpu.VMEM((2, 2, BIG), jnp.int32),
          pltpu.VMEM((2, 2, BIG), jnp.int32),
          pltpu.VMEM((4, BIG, feat), jnp.float32),
          pltpu.SemaphoreType.DMA,
          pltpu.SemaphoreType.DMA,
          pltpu.SemaphoreType.DMA,
          pltpu.SemaphoreType.DMA,
          pltpu.SemaphoreType.DMA,
          pltpu.SemaphoreType.DMA,
          pltpu.SemaphoreType.DMA,
          pltpu.SemaphoreType.DMA,
          pltpu.SemaphoreType.DMA,
          pltpu.SemaphoreType.DMA,
      ],
  )
  def prop_kernel(g_hbm, edges_hbm, out_hbm,
                  acc_sh, sidx_v, didx_v, rows_v,
                  isem, gsem0, gsem1, gsem2, gsem3,
                  ssem0, ssem1, ssem2, ssem3, zsem):
    cid = lax.axis_index("c")
    sid = lax.axis_index("s")
    wid = cid * 16 + sid
    gsem = (gsem0, gsem1, gsem2, gsem3)
    ssem = (ssem0, ssem1, ssem2, ssem3)
    base = sid * ROWS_PER_TILE

    for p in range(4):
      @pl.loop(0, BIG)
      def _zfill(r, _p=p):
        rows_v[_p, r, :] = jnp.zeros((feat,), jnp.float32)

    zcps = []
    for k in range(ROWS_PER_TILE // BIG):
      zcps.append(pltpu.async_copy(
          rows_v.at[k % 4],
          acc_sh.at[pl.ds(base + k * BIG, BIG)], zsem))
    for zcp in zcps:
      zcp.wait()
    plsc.subcore_barrier()
    pltpu.async_copy(edges_hbm.at[0, wid, 0], sidx_v.at[0], isem)
    pltpu.async_copy(edges_hbm.at[1, wid, 0], didx_v.at[0], isem)

    @pl.loop(0, groups, step=2)
    def _group(g):
      def idx(v, k):
        return v.at[k // 2, k % 2]

      pltpu.make_async_copy(edges_hbm.at[0, wid, g], sidx_v.at[0], isem).wait()
      pltpu.make_async_copy(edges_hbm.at[1, wid, g], didx_v.at[0], isem).wait()
      pltpu.async_copy(edges_hbm.at[0, wid, g + 1], sidx_v.at[1], isem)
      pltpu.async_copy(edges_hbm.at[1, wid, g + 1], didx_v.at[1], isem)
      gcp = [None] * 4
      scp = [None] * 4
      for k in range(4):
        if k == 2:
          pltpu.make_async_copy(
              edges_hbm.at[0, wid, g + 1], sidx_v.at[1], isem).wait()
          pltpu.make_async_copy(
              edges_hbm.at[1, wid, g + 1], didx_v.at[1], isem).wait()
        gcp[k] = pltpu.async_copy(
            g_hbm.at[idx(sidx_v, k)], rows_v.at[k], gsem[k])
        if k >= 1:
          gcp[k - 1].wait()
          scp[k - 1] = pltpu.async_copy(
              rows_v.at[k - 1], acc_sh.at[idx(didx_v, k - 1)], ssem[k - 1],
              add=True)
      gcp[3].wait()
      scp[3] = pltpu.async_copy(
          rows_v.at[3], acc_sh.at[idx(didx_v, 3)], ssem[3], add=True)
      scp[0].wait()
      scp[1].wait()
      nxt = jnp.minimum(g + 2, groups - 2)
      pltpu.async_copy(edges_hbm.at[0, wid, nxt], sidx_v.at[0], isem)
      pltpu.async_copy(edges_hbm.at[1, wid, nxt], didx_v.at[0], isem)
      scp[2].wait()
      scp[3].wait()

    pltpu.make_async_copy(edges_hbm.at[0, wid, 0], sidx_v.at[0], isem).wait()
    pltpu.make_async_copy(edges_hbm.at[1, wid, 0], didx_v.at[0], isem).wait()
    plsc.subcore_barrier()
    pltpu.sync_copy(acc_sh.at[pl.ds(base, ROWS_PER_TILE)],
                    out_hbm.at[cid, pl.ds(base, ROWS_PER_TILE)])

  return prop_kernel


def _row_spec(feat):
  return pl.BlockSpec((BT, feat), lambda i: (i, 0))


def _pair_spec(*feat):
  if feat:
    return pl.BlockSpec((2, BT, feat[0]), lambda i: (0, i, 0))
  return pl.BlockSpec((2, BT), lambda i: (0, i))


def _full_spec(shape):
  return pl.BlockSpec(shape, lambda i: (0,) * len(shape))


def _tc1_body(d, x, w1, dinv_o, t1_o, g1_o):
  deg = (d[0, :] + d[1, :] + 1.0).reshape(BT, 1)
  dinv = lax.rsqrt(deg)
  dinv_o[...] = dinv
  t1 = jnp.dot(x[...], w1[...], preferred_element_type=jnp.float32)
  t1_o[...] = t1
  g1_o[...] = t1 * dinv


def _tc2_body(s, t1, dinv, b1, h1_o, g2_o):
  dv = dinv[...]
  agg = dv * (s[0] + s[1]) + dv * dv * t1[...] + b1[...]
  h1 = jnp.maximum(agg, 0.0)
  h1_o[...] = h1
  g2_o[...] = h1 * dv


def _tc3_body(s, h1, dinv, w2, b2, wl, bl, out_o):
  dv = dinv[...]
  agg = dv * (s[0] + s[1]) + dv * dv * h1[...]
  h2 = jnp.dot(agg, w2[...], preferred_element_type=jnp.float32) + b2[...]
  h2 = jnp.maximum(h2, 0.0)
  z = jnp.dot(h2, wl[...], preferred_element_type=jnp.float32) + bl[...]
  out_o[...] = jax.nn.sigmoid(z)


def kernel(x, edge_index, W1, b1, W2, b2, Wl, bl):
  n = x.shape[0]
  e = edge_index.shape[1]
  groups = -(-e // (NW * 2 * BIG))
  groups += groups % 2
  ep = NW * groups * 2 * BIG
  edges = jnp.pad(edge_index, ((0, 0), (0, ep - e)), constant_values=n)
  edges = edges.reshape(2, NW, groups, 2, BIG)
  xp = jnp.zeros((NP, 4), jnp.float32).at[:n].set(x)

  grid = (NP // BT,)

  degs = _make_deg_kernel(groups)(edges)

  dinv, t1, g1 = pl.pallas_call(
      _tc1_body,
      grid=grid,
      in_specs=[_pair_spec(), _row_spec(4), _full_spec((4, 16))],
      out_specs=[_row_spec(1), _row_spec(16), _row_spec(16)],
      out_shape=[jax.ShapeDtypeStruct((NP, 1), jnp.float32),
                 jax.ShapeDtypeStruct((NP, 16), jnp.float32),
                 jax.ShapeDtypeStruct((NP, 16), jnp.float32)],
  )(degs, xp, W1)

  s1 = _make_prop_kernel(groups, 16)(g1, edges)

  h1, g2 = pl.pallas_call(
      _tc2_body,
      grid=grid,
      in_specs=[_pair_spec(16), _row_spec(16), _row_spec(1),
                _full_spec((1, 16))],
      out_specs=[_row_spec(16), _row_spec(16)],
      out_shape=[jax.ShapeDtypeStruct((NP, 16), jnp.float32),
                 jax.ShapeDtypeStruct((NP, 16), jnp.float32)],
  )(s1, t1, dinv, b1.reshape(1, 16))

  s2 = _make_prop_kernel(groups, 16)(g2, edges)

  out = pl.pallas_call(
      _tc3_body,
      grid=grid,
      in_specs=[_pair_spec(16), _row_spec(16), _row_spec(1),
                _full_spec((16, 32)), _full_spec((1, 32)),
                _full_spec((32, 1)), _full_spec((1, 1))],
      out_specs=_row_spec(1),
      out_shape=jax.ShapeDtypeStruct((n, 1), jnp.float32),
  )(s2, h1, dinv, W2, b2.reshape(1, 32), Wl, bl.reshape(1, 1))

  return out


# speedup vs baseline: 1.4145x; 1.0242x over previous
import functools

import jax
import jax.numpy as jnp
from jax import lax
from jax.experimental import pallas as pl
from jax.experimental.pallas import tpu as pltpu
from jax.experimental.pallas import tpu_sc as plsc

NP = 102400
ROWS_PER_TILE = NP // 16
NW = 32
BIG = 256
BT = 6400


def _sc_mesh():
  return plsc.VectorSubcoreMesh(core_axis_name="c", subcore_axis_name="s")


@functools.lru_cache(maxsize=None)
def _make_deg_kernel(groups):
  @functools.partial(
      pl.kernel,
      mesh=_sc_mesh(),
      out_type=jax.ShapeDtypeStruct((2, NP), jnp.float32),
      compiler_params=pltpu.CompilerParams(use_tc_tiling_on_sc=False),
      scratch_types=[
          pltpu.VMEM_SHARED((NP,), jnp.float32),
          pltpu.VMEM((2, 2, BIG), jnp.int32),
          pltpu.VMEM((BIG,), jnp.float32),
          pltpu.VMEM((640,), jnp.float32),
          pltpu.SemaphoreType.DMA,
          pltpu.SemaphoreType.DMA,
          pltpu.SemaphoreType.DMA,
      ],
  )
  def deg_kernel(edges_hbm, out_hbm, acc_sh, idx_v, ones_v, zbuf_v,
                 isem, ssem, zsem):
    cid = lax.axis_index("c")
    sid = lax.axis_index("s")
    wid = cid * 16 + sid
    @pl.loop(0, BIG // 16)
    def _ofill(i):
      ones_v[pl.ds(i * 16, 16)] = jnp.ones((16,), jnp.float32)

    @pl.loop(0, 40)
    def _zfill(i):
      zbuf_v[pl.ds(i * 16, 16)] = jnp.zeros((16,), jnp.float32)

    base = sid * ROWS_PER_TILE
    zcps = [pltpu.async_copy(zbuf_v, acc_sh.at[pl.ds(base + k * 640, 640)],
                             zsem) for k in range(ROWS_PER_TILE // 640)]
    for zcp in zcps:
      zcp.wait()
    plsc.subcore_barrier()
    pltpu.async_copy(edges_hbm.at[1, wid, 0], idx_v.at[0], isem)

    @pl.loop(0, groups, step=2)
    def _group(g):
      for b in range(2):
        cur = g + b
        nb = 1 - b
        pltpu.make_async_copy(
            edges_hbm.at[1, wid, cur], idx_v.at[b], isem).wait()
        nxt = jnp.minimum(cur + 1, groups - 1)
        pltpu.async_copy(edges_hbm.at[1, wid, nxt], idx_v.at[nb], isem)
        scps = []
        for j in range(2):
          scps.append(pltpu.async_copy(
              ones_v, acc_sh.at[idx_v.at[b, j]], ssem, add=True))
        for scp in scps:
          scp.wait()

    pltpu.make_async_copy(edges_hbm.at[1, wid, 0], idx_v.at[0], isem).wait()
    plsc.subcore_barrier()
    pltpu.sync_copy(acc_sh.at[pl.ds(base, ROWS_PER_TILE)],
                    out_hbm.at[cid, pl.ds(base, ROWS_PER_TILE)])

  return deg_kernel


@functools.lru_cache(maxsize=None)
def _make_prop_kernel(groups, feat):
  @functools.partial(
      pl.kernel,
      mesh=_sc_mesh(),
      out_type=jax.ShapeDtypeStruct((2, NP, feat), jnp.float32),
      compiler_params=pltpu.CompilerParams(use_tc_tiling_on_sc=False),
      scratch_types=[
          pltpu.VMEM_SHARED((NP, feat), jnp.float32),
          pltpu.VMEM((2, 2, BIG), jnp.int32),
          pltpu.VMEM((2, 2, BIG), jnp.int32),
          pltpu.VMEM((4, BIG, feat), jnp.float32),
          pltpu.SemaphoreType.DMA,
          pltpu.SemaphoreType.DMA,
          pltpu.SemaphoreType.DMA,
          pltpu.SemaphoreType.DMA,
          pltpu.SemaphoreType.DMA,
          pltpu.SemaphoreType.DMA,
          pltpu.SemaphoreType.DMA,
          pltpu.SemaphoreType.DMA,
          pltpu.SemaphoreType.DMA,
          pltpu.SemaphoreType.DMA,
      ],
  )
  def prop_kernel(g_hbm, edges_hbm, out_hbm,
                  acc_sh, sidx_v, didx_v, rows_v,
                  isem, gsem0, gsem1, gsem2, gsem3,
                  ssem0, ssem1, ssem2, ssem3, zsem):
    cid = lax.axis_index("c")
    sid = lax.axis_index("s")
    wid = cid * 16 + sid
    gsem = (gsem0, gsem1, gsem2, gsem3)
    ssem = (ssem0, ssem1, ssem2, ssem3)
    base = sid * ROWS_PER_TILE

    for p in range(4):
      @pl.loop(0, BIG)
      def _zfill(r, _p=p):
        rows_v[_p, r, :] = jnp.zeros((feat,), jnp.float32)

    zcps = []
    for k in range(ROWS_PER_TILE // BIG):
      zcps.append(pltpu.async_copy(
          rows_v.at[k % 4],
          acc_sh.at[pl.ds(base + k * BIG, BIG)], zsem))
    for zcp in zcps:
      zcp.wait()
    plsc.subcore_barrier()
    pltpu.async_copy(edges_hbm.at[0, wid, 0], sidx_v.at[0], isem)
    pltpu.async_copy(edges_hbm.at[1, wid, 0], didx_v.at[0], isem)

    @pl.loop(0, groups, step=2)
    def _group(g):
      def idx(v, k):
        return v.at[k // 2, k % 2]

      pltpu.make_async_copy(edges_hbm.at[0, wid, g], sidx_v.at[0], isem).wait()
      pltpu.make_async_copy(edges_hbm.at[1, wid, g], didx_v.at[0], isem).wait()
      pltpu.async_copy(edges_hbm.at[0, wid, g + 1], sidx_v.at[1], isem)
      pltpu.async_copy(edges_hbm.at[1, wid, g + 1], didx_v.at[1], isem)
      gcp = [None] * 4
      scp = [None] * 4
      for k in range(4):
        if k == 2:
          pltpu.make_async_copy(
              edges_hbm.at[0, wid, g + 1], sidx_v.at[1], isem).wait()
          pltpu.make_async_copy(
              edges_hbm.at[1, wid, g + 1], didx_v.at[1], isem).wait()
        gcp[k] = pltpu.async_copy(
            g_hbm.at[idx(sidx_v, k)], rows_v.at[k], gsem[k])
        if k >= 1:
          gcp[k - 1].wait()
          scp[k - 1] = pltpu.async_copy(
              rows_v.at[k - 1], acc_sh.at[idx(didx_v, k - 1)], ssem[k - 1],
              add=True)
      gcp[3].wait()
      scp[3] = pltpu.async_copy(
          rows_v.at[3], acc_sh.at[idx(didx_v, 3)], ssem[3], add=True)
      scp[0].wait()
      scp[1].wait()
      nxt = jnp.minimum(g + 2, groups - 2)
      pltpu.async_copy(edges_hbm.at[0, wid, nxt], sidx_v.at[0], isem)
      pltpu.async_copy(edges_hbm.at[1, wid, nxt], didx_v.at[0], isem)
      scp[2].wait()
      scp[3].wait()

    pltpu.make_async_copy(edges_hbm.at[0, wid, 0], sidx_v.at[0], isem).wait()
    pltpu.make_async_copy(edges_hbm.at[1, wid, 0], didx_v.at[0], isem).wait()
    plsc.subcore_barrier()
    pltpu.sync_copy(acc_sh.at[pl.ds(base, ROWS_PER_TILE)],
                    out_hbm.at[cid, pl.ds(base, ROWS_PER_TILE)])

  return prop_kernel


def _row_spec(feat):
  return pl.BlockSpec((BT, feat), lambda i: (i, 0))


def _pair_spec(*feat):
  if feat:
    return pl.BlockSpec((2, BT, feat[0]), lambda i: (0, i, 0))
  return pl.BlockSpec((2, BT), lambda i: (0, i))


def _full_spec(shape):
  return pl.BlockSpec(shape, lambda i: (0,) * len(shape))


def _tc1_body(d, x, w1, dinv_o, t1_o, g1_o):
  deg = (d[0, :] + d[1, :] + 1.0).reshape(BT, 1)
  dinv = lax.rsqrt(deg)
  dinv_o[...] = dinv
  t1 = jnp.dot(x[...], w1[...], preferred_element_type=jnp.float32)
  t1_o[...] = t1
  g1_o[...] = t1 * dinv


def _tc2_body(s, t1, dinv, b1, h1_o, g2_o):
  dv = dinv[...]
  agg = dv * (s[0] + s[1]) + dv * dv * t1[...] + b1[...]
  h1 = jnp.maximum(agg, 0.0)
  h1_o[...] = h1
  g2_o[...] = h1 * dv


def _tc3_body(s, h1, dinv, w2, b2, wl, bl, out_o):
  dv = dinv[...]
  agg = dv * (s[0] + s[1]) + dv * dv * h1[...]
  h2 = jnp.dot(agg, w2[...], preferred_element_type=jnp.float32) + b2[...]
  h2 = jnp.maximum(h2, 0.0)
  z = jnp.dot(h2, wl[...], preferred_element_type=jnp.float32) + bl[...]
  out_o[...] = jax.nn.sigmoid(z)


def kernel(x, edge_index, W1, b1, W2, b2, Wl, bl):
  n = x.shape[0]
  e = edge_index.shape[1]
  groups = -(-e // (NW * 2 * BIG))
  groups += groups % 2
  ep = NW * groups * 2 * BIG
  edges = jnp.pad(edge_index, ((0, 0), (0, ep - e)), constant_values=n)
  edges = edges.reshape(2, NW, groups, 2, BIG)

  grid = (NP // BT,)

  degs = _make_deg_kernel(groups)(edges)

  dinv, t1, g1 = pl.pallas_call(
      _tc1_body,
      grid=grid,
      in_specs=[_pair_spec(), _row_spec(4), _full_spec((4, 16))],
      out_specs=[_row_spec(1), _row_spec(16), _row_spec(16)],
      out_shape=[jax.ShapeDtypeStruct((NP, 1), jnp.float32),
                 jax.ShapeDtypeStruct((NP, 16), jnp.float32),
                 jax.ShapeDtypeStruct((NP, 16), jnp.float32)],
  )(degs, x, W1)

  s1 = _make_prop_kernel(groups, 16)(g1, edges)

  h1, g2 = pl.pallas_call(
      _tc2_body,
      grid=grid,
      in_specs=[_pair_spec(16), _row_spec(16), _row_spec(1),
                _full_spec((1, 16))],
      out_specs=[_row_spec(16), _row_spec(16)],
      out_shape=[jax.ShapeDtypeStruct((NP, 16), jnp.float32),
                 jax.ShapeDtypeStruct((NP, 16), jnp.float32)],
  )(s1, t1, dinv, b1.reshape(1, 16))

  s2 = _make_prop_kernel(groups, 16)(g2, edges)

  out = pl.pallas_call(
      _tc3_body,
      grid=grid,
      in_specs=[_pair_spec(16), _row_spec(16), _row_spec(1),
                _full_spec((16, 32)), _full_spec((1, 32)),
                _full_spec((32, 1)), _full_spec((1, 1))],
      out_specs=_row_spec(1),
      out_shape=jax.ShapeDtypeStruct((n, 1), jnp.float32),
  )(s2, h1, dinv, W2, b2.reshape(1, 32), Wl, bl.reshape(1, 1))

  return out
